# Initial kernel scaffold; baseline (speedup 1.0000x reference)
import jax
import jax.numpy as jnp
from jax.experimental import pallas as pl


def kernel(weights, existing_bins):
    raise NotImplementedError("write your pallas kernel here")


# speedup vs baseline: 4.5854x; 4.5854x over previous
import functools

import jax
import jax.numpy as jnp
import numpy as np
from jax import lax
from jax.experimental import pallas as pl
from jax.experimental.pallas import tpu as pltpu
from jax.experimental.pallas import tpu_sc as plsc

B = 16384
N = 128
NB = N + 1
J = 65
EPS = 1e-5
NEAR = 0.05
FAR = 6.0

NUM_CORES = 2
NUM_SUBCORES = 16
NW = NUM_CORES * NUM_SUBCORES
RAYS_PER_W = B // NW
C = 128
G = C // 16
CHUNKS = RAYS_PER_W // C

_mesh = plsc.VectorSubcoreMesh(core_axis_name="c", subcore_axis_name="s")

_f32 = jnp.float32
_i32 = jnp.int32


def _body(w_hbm, eb_hbm, u_hbm,
          bs_hbm, be_hbm, ss_hbm, se_hbm,
          wbuf, ebbuf, cdfbuf, mbuf, ubuf,
          obs, obe, oss, ose):
    wid = lax.axis_index("s") * NUM_CORES + lax.axis_index("c")
    lane = lax.broadcasted_iota(_i32, (16,), 0)
    zf = jnp.zeros((16,), _f32)
    zi = jnp.zeros((16,), _i32)

    pltpu.sync_copy(u_hbm, ubuf)

    def chunk_body(ci, _):
        base = wid * RAYS_PER_W + ci * C
        pltpu.sync_copy(w_hbm.at[pl.ds(base, C)], wbuf)
        pltpu.sync_copy(eb_hbm.at[pl.ds(base, C)], ebbuf)

        def group_body(g, _):
            rowv = g * 16 + lane

            cdfbuf[0, :] = zf

            def pa(i, acc):
                wv = plsc.load_gather(wbuf, [rowv, jnp.full((16,), i, _i32)])
                acc = acc + wv
                cdfbuf[i + 1, :] = acc
                return acc
            total = lax.fori_loop(0, N, pa, zf)

            pad = jnp.maximum(EPS - total, 0.0)
            inv = 1.0 / (total + pad)
            padper = pad * (1.0 / N)

            def mi(j, _):
                mbuf[j, :] = zi
                return 0
            lax.fori_loop(0, J + 1, mi, 0)

            def pb(i, ppi):
                raw = cdfbuf[i + 1, :]
                c = jnp.minimum((raw + ppi) * inv, 1.0)
                cdfbuf[i + 1, :] = c
                k = jnp.clip((c * float(J) + 0.5).astype(_i32), 0, J)
                plsc.store_scatter(mbuf, [k, lane],
                                   jnp.full((16,), i + 1, _i32))
                return ppi + padper
            lax.fori_loop(0, N, pb, padper)

            def pc(j, run):
                run = jnp.maximum(run, mbuf[j, :])
                below = run
                above = jnp.minimum(below + 1, N)
                c0 = plsc.load_gather(cdfbuf, [below, lane])
                c1 = plsc.load_gather(cdfbuf, [above, lane])
                e0 = plsc.load_gather(ebbuf, [rowv, below])
                e1 = plsc.load_gather(ebbuf, [rowv, above])
                uu = ubuf[j, :]
                denom = c1 - c0
                denom = jnp.where(denom < 1e-5, 1.0, denom)
                t = jnp.clip((uu - c0) / denom, 0.0, 1.0)
                binsv = e0 + t * (e1 - e0)
                eucl = NEAR + binsv * (FAR - NEAR)
                colj = jnp.full((16,), j, _i32)
                mlo = colj < (J - 1)
                mhi = colj > 0
                cols = jnp.minimum(colj, J - 2)
                cole = jnp.maximum(colj - 1, 0)
                plsc.store_scatter(oss, [rowv, cols], binsv, mask=mlo)
                plsc.store_scatter(obs, [rowv, cols], eucl, mask=mlo)
                plsc.store_scatter(ose, [rowv, cole], binsv, mask=mhi)
                plsc.store_scatter(obe, [rowv, cole], eucl, mask=mhi)
                return run
            lax.fori_loop(0, J, pc, zi)
            return 0
        lax.fori_loop(0, G, group_body, 0)

        pltpu.sync_copy(obs, bs_hbm.at[pl.ds(base, C)])
        pltpu.sync_copy(obe, be_hbm.at[pl.ds(base, C)])
        pltpu.sync_copy(oss, ss_hbm.at[pl.ds(base, C)])
        pltpu.sync_copy(ose, se_hbm.at[pl.ds(base, C)])
        return 0
    lax.fori_loop(0, CHUNKS, chunk_body, 0)


_sampler = functools.partial(
    pl.kernel,
    mesh=_mesh,
    compiler_params=pltpu.CompilerParams(
        needs_layout_passes=False, use_tc_tiling_on_sc=False),
    out_type=[jax.ShapeDtypeStruct((B, J - 1), _f32)] * 4,
    scratch_types=[
        pltpu.VMEM((C, N), _f32),
        pltpu.VMEM((C, NB), _f32),
        pltpu.VMEM((NB, 16), _f32),
        pltpu.VMEM((J + 1, 16), _i32),
        pltpu.VMEM((J, 16), _f32),
        pltpu.VMEM((C, J - 1), _f32),
        pltpu.VMEM((C, J - 1), _f32),
        pltpu.VMEM((C, J - 1), _f32),
        pltpu.VMEM((C, J - 1), _f32),
    ],
)(_body)


def _u_table():
    u = jnp.linspace(0.0, 1.0 - 1.0 / J, J, dtype=_f32) + _f32(1.0 / (2 * J))
    return jnp.broadcast_to(u[:, None], (J, 16)).astype(_f32)


def kernel(weights, existing_bins):
    w2d = weights.reshape(B, N)
    u_tab = _u_table()
    bs, be, ss, se = _sampler(w2d, existing_bins, u_tab)
    return (bs[:, :, None], be[:, :, None], ss[:, :, None], se[:, :, None])


# speedup vs baseline: 7.6841x; 1.6758x over previous
import functools

import jax
import jax.numpy as jnp
from jax import lax
from jax.experimental import pallas as pl
from jax.experimental.pallas import tpu as pltpu
from jax.experimental.pallas import tpu_sc as plsc

B = 16384
N = 128
NB = N + 1
J = 65
NO = J - 1
EPS = 1e-5
NEAR = 0.05
FAR = 6.0

NUM_CORES = 2
NUM_SUBCORES = 16
NW = NUM_CORES * NUM_SUBCORES
RAYS_PER_W = B // NW
C = 128
G = C // 16
CHUNKS = RAYS_PER_W // C

_mesh = plsc.VectorSubcoreMesh(core_axis_name="c", subcore_axis_name="s")

_f32 = jnp.float32
_i32 = jnp.int32


def _body(w_hbm, eb_hbm, u_hbm,
          bs_hbm, be_hbm, ss_hbm, se_hbm,
          wbuf, ebbuf, cdfbuf, mbuf, ubuf,
          obs, obe, oss, ose):
    wid = lax.axis_index("s") * NUM_CORES + lax.axis_index("c")
    lane = lax.broadcasted_iota(_i32, (16,), 0)
    cdfbase = lane * NB
    zf = jnp.zeros((16,), _f32)
    zi = jnp.zeros((16,), _i32)
    ones_i = jnp.ones((16,), _i32)

    pltpu.sync_copy(u_hbm, ubuf)

    @plsc.parallel_loop(0, J + 1, unroll=6)
    def _minit(j):
        mbuf[j, :] = zi

    def chunk_body(ci, _):
        base = wid * RAYS_PER_W + ci * C
        pltpu.sync_copy(w_hbm.at[pl.ds(base * N, C * N)], wbuf)
        pltpu.sync_copy(eb_hbm.at[pl.ds(base * NB, C * NB)], ebbuf)

        def group_body(g, _):
            rows = g * 16 + lane
            wbase = rows * N
            ebbase = rows * NB
            obase = rows * NO

            plsc.store_scatter(cdfbuf, [cdfbase], zf)

            @plsc.parallel_loop(0, N, unroll=8, carry=zf)
            def total(i, acc):
                wv = plsc.load_gather(wbuf, [wbase + i])
                acc = acc + wv
                plsc.store_scatter(cdfbuf, [cdfbase + (i + 1)], acc)
                return acc

            pad = jnp.maximum(EPS - total, 0.0)
            inv = 1.0 / (total + pad)
            padper = pad * (1.0 / N)

            @plsc.parallel_loop(0, N, unroll=8)
            def _pb(i):
                idx = cdfbase + (i + 1)
                raw = plsc.load_gather(cdfbuf, [idx])
                fi = (i + 1).astype(_f32)
                c = jnp.minimum((raw + padper * fi) * inv, 1.0)
                plsc.store_scatter(cdfbuf, [idx], c)
                k = (c * float(J) + 0.5).astype(_i32)
                plsc.addupdate_scatter(mbuf, [k, lane], ones_i)

            def interp(j, below):
                above = jnp.minimum(below + 1, N)
                c0 = plsc.load_gather(cdfbuf, [cdfbase + below])
                c1 = plsc.load_gather(cdfbuf, [cdfbase + above])
                e0 = plsc.load_gather(ebbuf, [ebbase + below])
                e1 = plsc.load_gather(ebbuf, [ebbase + above])
                uu = ubuf[j, :]
                denom = c1 - c0
                denom = jnp.where(denom < 1e-5, 1.0, denom)
                t = jnp.clip((uu - c0) / denom, 0.0, 1.0)
                binsv = e0 + t * (e1 - e0)
                eucl = NEAR + binsv * (FAR - NEAR)
                return binsv, eucl

            run0 = mbuf[0, :]
            mbuf[0, :] = zi
            binsv, eucl = interp(0, run0)
            plsc.store_scatter(oss, [obase], binsv)
            plsc.store_scatter(obs, [obase], eucl)

            @plsc.parallel_loop(1, J - 1, unroll=7, carry=run0)
            def runf(j, run):
                run = run + mbuf[j, :]
                mbuf[j, :] = zi
                binsv, eucl = interp(j, run)
                oe = obase + (j - 1)
                os_ = obase + j
                plsc.store_scatter(ose, [oe], binsv)
                plsc.store_scatter(obe, [oe], eucl)
                plsc.store_scatter(oss, [os_], binsv)
                plsc.store_scatter(obs, [os_], eucl)
                return run

            runl = runf + mbuf[J - 1, :]
            mbuf[J - 1, :] = zi
            mbuf[J, :] = zi
            binsv, eucl = interp(J - 1, runl)
            plsc.store_scatter(ose, [obase + (NO - 1)], binsv)
            plsc.store_scatter(obe, [obase + (NO - 1)], eucl)
            return 0
        lax.fori_loop(0, G, group_body, 0)

        pltpu.sync_copy(obs, bs_hbm.at[pl.ds(base * NO, C * NO)])
        pltpu.sync_copy(obe, be_hbm.at[pl.ds(base * NO, C * NO)])
        pltpu.sync_copy(oss, ss_hbm.at[pl.ds(base * NO, C * NO)])
        pltpu.sync_copy(ose, se_hbm.at[pl.ds(base * NO, C * NO)])
        return 0
    lax.fori_loop(0, CHUNKS, chunk_body, 0)


_sampler = functools.partial(
    pl.kernel,
    mesh=_mesh,
    compiler_params=pltpu.CompilerParams(
        needs_layout_passes=False, use_tc_tiling_on_sc=False),
    out_type=[jax.ShapeDtypeStruct((B * NO,), _f32)] * 4,
    scratch_types=[
        pltpu.VMEM((C * N,), _f32),
        pltpu.VMEM((C * NB,), _f32),
        pltpu.VMEM((16 * NB,), _f32),
        pltpu.VMEM((J + 1, 16), _i32),
        pltpu.VMEM((J, 16), _f32),
        pltpu.VMEM((C * NO,), _f32),
        pltpu.VMEM((C * NO,), _f32),
        pltpu.VMEM((C * NO,), _f32),
        pltpu.VMEM((C * NO,), _f32),
    ],
)(_body)


def _u_table():
    u = jnp.linspace(0.0, 1.0 - 1.0 / J, J, dtype=_f32) + _f32(1.0 / (2 * J))
    return jnp.broadcast_to(u[:, None], (J, 16)).astype(_f32)


def kernel(weights, existing_bins):
    wf = weights.reshape(B * N)
    ebf = existing_bins.reshape(B * NB)
    u_tab = _u_table()
    bs, be, ss, se = _sampler(wf, ebf, u_tab)
    shp = (B, NO, 1)
    return (bs.reshape(shp), be.reshape(shp), ss.reshape(shp), se.reshape(shp))


# speedup vs baseline: 11.7528x; 1.5295x over previous
import functools

import jax
import jax.numpy as jnp
from jax import lax
from jax.experimental import pallas as pl
from jax.experimental.pallas import tpu as pltpu
from jax.experimental.pallas import tpu_sc as plsc

B = 16384
N = 128
NB = N + 1
J = 65
NO = J - 1
EPS = 1e-5
NEAR = 0.05
FAR = 6.0

NUM_CORES = 2
NUM_SUBCORES = 16
NW = NUM_CORES * NUM_SUBCORES
RAYS_PER_W = B // NW
C = 128
G = C // 16
CHUNKS = RAYS_PER_W // C

_mesh = plsc.VectorSubcoreMesh(core_axis_name="c", subcore_axis_name="s")

_f32 = jnp.float32
_i32 = jnp.int32


def _body(w_hbm, eb_hbm, u_hbm,
          bs_hbm, be_hbm, ss_hbm, se_hbm,
          wbuf, ebbuf, cdfbuf, mbuf, ubuf,
          obs, obe, oss, ose):
    wid = lax.axis_index("s") * NUM_CORES + lax.axis_index("c")
    lane = lax.broadcasted_iota(_i32, (16,), 0)
    cdfbase = lane * NB
    zf = jnp.zeros((16,), _f32)
    zi = jnp.zeros((16,), _i32)
    ones_i = jnp.ones((16,), _i32)

    pltpu.sync_copy(u_hbm, ubuf)

    @plsc.parallel_loop(0, J + 1, unroll=6)
    def _minit(j):
        mbuf[j, :] = zi

    def chunk_body(ci, _):
        base = wid * RAYS_PER_W + ci * C
        pltpu.sync_copy(w_hbm.at[pl.ds(base * N, C * N)], wbuf)
        pltpu.sync_copy(eb_hbm.at[pl.ds(base * NB, C * NB)], ebbuf)

        def group_body(g, _):
            rows = g * 16 + lane
            wbase = rows * N
            ebbase = rows * NB
            obase = rows * NO

            plsc.store_scatter(cdfbuf, [cdfbase], zf)

            @plsc.parallel_loop(0, N, unroll=8, carry=zf)
            def total(i, acc):
                wv = plsc.load_gather(wbuf, [wbase + i])
                acc = acc + wv
                plsc.store_scatter(cdfbuf, [cdfbase + (i + 1)], acc)
                return acc

            pad = jnp.maximum(EPS - total, 0.0)
            inv = 1.0 / (total + pad)
            padper = pad * (1.0 / N)

            @plsc.parallel_loop(0, N, unroll=8)
            def _pb(i):
                idx = cdfbase + (i + 1)
                raw = plsc.load_gather(cdfbuf, [idx])
                fi = (i + 1).astype(_f32)
                c = jnp.minimum((raw + padper * fi) * inv, 1.0)
                plsc.store_scatter(cdfbuf, [idx], c)
                k = (c * float(J) + 0.5).astype(_i32)
                plsc.addupdate_scatter(mbuf, [k, lane], ones_i)

            def interp(j, below):
                above = jnp.minimum(below + 1, N)
                c0 = plsc.load_gather(cdfbuf, [cdfbase + below])
                c1 = plsc.load_gather(cdfbuf, [cdfbase + above])
                e0 = plsc.load_gather(ebbuf, [ebbase + below])
                e1 = plsc.load_gather(ebbuf, [ebbase + above])
                uu = ubuf[j, :]
                denom = c1 - c0
                denom = jnp.where(denom < 1e-5, 1.0, denom)
                t = jnp.clip((uu - c0) / denom, 0.0, 1.0)
                binsv = e0 + t * (e1 - e0)
                eucl = NEAR + binsv * (FAR - NEAR)
                return binsv, eucl

            run0 = mbuf[0, :]
            mbuf[0, :] = zi
            binsv, eucl = interp(0, run0)
            crow = rows
            zrow = jnp.zeros((16,), _i32)
            plsc.store_scatter(oss, [zrow, crow], binsv)
            plsc.store_scatter(obs, [zrow, crow], eucl)

            @plsc.parallel_loop(1, J - 1, unroll=7, carry=run0)
            def runf(j, run):
                run = run + mbuf[j, :]
                mbuf[j, :] = zi
                binsv, eucl = interp(j, run)
                oe = jnp.full((16,), j - 1, _i32)
                os_ = oe + 1
                plsc.store_scatter(ose, [oe, crow], binsv)
                plsc.store_scatter(obe, [oe, crow], eucl)
                plsc.store_scatter(oss, [os_, crow], binsv)
                plsc.store_scatter(obs, [os_, crow], eucl)
                return run

            runl = runf + mbuf[J - 1, :]
            mbuf[J - 1, :] = zi
            mbuf[J, :] = zi
            binsv, eucl = interp(J - 1, runl)
            lrow = jnp.full((16,), NO - 1, _i32)
            plsc.store_scatter(ose, [lrow, crow], binsv)
            plsc.store_scatter(obe, [lrow, crow], eucl)
            return 0
        lax.fori_loop(0, G, group_body, 0)

        pltpu.sync_copy(obs, bs_hbm.at[:, pl.ds(base, C)])
        pltpu.sync_copy(obe, be_hbm.at[:, pl.ds(base, C)])
        pltpu.sync_copy(oss, ss_hbm.at[:, pl.ds(base, C)])
        pltpu.sync_copy(ose, se_hbm.at[:, pl.ds(base, C)])
        return 0
    lax.fori_loop(0, CHUNKS, chunk_body, 0)


_sampler = functools.partial(
    pl.kernel,
    mesh=_mesh,
    compiler_params=pltpu.CompilerParams(
        needs_layout_passes=False, use_tc_tiling_on_sc=False),
    out_type=[jax.ShapeDtypeStruct((NO, B), _f32)] * 4,
    scratch_types=[
        pltpu.VMEM((C * N,), _f32),
        pltpu.VMEM((C * NB,), _f32),
        pltpu.VMEM((16 * NB,), _f32),
        pltpu.VMEM((J + 1, 16), _i32),
        pltpu.VMEM((J, 16), _f32),
        pltpu.VMEM((NO, C), _f32),
        pltpu.VMEM((NO, C), _f32),
        pltpu.VMEM((NO, C), _f32),
        pltpu.VMEM((NO, C), _f32),
    ],
)(_body)


def _u_table():
    u = jnp.linspace(0.0, 1.0 - 1.0 / J, J, dtype=_f32) + _f32(1.0 / (2 * J))
    return jnp.broadcast_to(u[:, None], (J, 16)).astype(_f32)


def kernel(weights, existing_bins):
    wf = weights.reshape(B * N)
    ebf = existing_bins.reshape(B * NB)
    u_tab = _u_table()
    bs, be, ss, se = _sampler(wf, ebf, u_tab)
    def _t(x):
        return jnp.transpose(x)[:, :, None]
    return (_t(bs), _t(be), _t(ss), _t(se))


# speedup vs baseline: 14.0555x; 1.1959x over previous
import functools

import jax
import jax.numpy as jnp
from jax import lax
from jax.experimental import pallas as pl
from jax.experimental.pallas import tpu as pltpu
from jax.experimental.pallas import tpu_sc as plsc

B = 16384
N = 128
NB = N + 1
J = 65
NO = J - 1
EPS = 1e-5
NEAR = 0.05
FAR = 6.0

NUM_CORES = 2
NUM_SUBCORES = 16
NW = NUM_CORES * NUM_SUBCORES
RAYS_PER_W = B // NW
C = 128
G = C // 16
CHUNKS = RAYS_PER_W // C

_mesh = plsc.VectorSubcoreMesh(core_axis_name="c", subcore_axis_name="s")

_f32 = jnp.float32
_i32 = jnp.int32


def _body(w_hbm, ebt_hbm,
          bs_hbm, be_hbm, ss_hbm, se_hbm,
          wbuf, ebbuf, cdfbuf, mbuf, binsbuf, euclbuf):
    wid = lax.axis_index("s") * NUM_CORES + lax.axis_index("c")
    lane = lax.broadcasted_iota(_i32, (16,), 0)
    zf = jnp.zeros((16,), _f32)
    zi = jnp.zeros((16,), _i32)
    ones_i = jnp.ones((16,), _i32)

    @plsc.parallel_loop(0, J + 1, unroll=6)
    def _minit(j):
        mbuf[j, :] = zi

    def chunk_body(ci, _):
        base = wid * RAYS_PER_W + ci * C
        pltpu.sync_copy(w_hbm.at[pl.ds(base * N, C * N)], wbuf)
        pltpu.sync_copy(ebt_hbm.at[:, pl.ds(base, C)], ebbuf)

        def group_body(g, _):
            crow = g * 16 + lane
            wbase = crow * N

            cdfbuf[0, :] = zf

            @plsc.parallel_loop(0, N, step=8, unroll=2, carry=zf)
            def total(i, acc):
                ib = wbase + i
                w = [plsc.load_gather(wbuf, [ib + k]) for k in range(8)]
                s01 = w[0] + w[1]
                s23 = w[2] + w[3]
                s45 = w[4] + w[5]
                s67 = w[6] + w[7]
                s03 = s01 + s23
                s47 = s45 + s67
                p = [w[0], s01, s01 + w[2], s03, s03 + w[4], s03 + s45,
                     s03 + s45 + w[6], s03 + s47]
                for k in range(8):
                    cdfbuf[i + 1 + k, :] = acc + p[k]
                return acc + p[7]

            pad = jnp.maximum(EPS - total, 0.0)
            inv = 1.0 / (total + pad)
            padper = pad * (1.0 / N)

            @plsc.parallel_loop(0, N, unroll=8)
            def _pb(i):
                raw = cdfbuf[i + 1, :]
                fi = (i + 1).astype(_f32)
                c = jnp.minimum((raw + padper * fi) * inv, 1.0)
                cdfbuf[i + 1, :] = c
                k = (c * float(J) + 0.5).astype(_i32)
                plsc.addupdate_scatter(mbuf, [k, lane], ones_i)

            col0 = g * 16

            @plsc.parallel_loop(0, J, unroll=5, carry=zi)
            def _run(j, run):
                run = run + mbuf[j, :]
                mbuf[j, :] = zi
                below = run
                above = jnp.minimum(below + 1, N)
                c0 = plsc.load_gather(cdfbuf, [below, lane])
                c1 = plsc.load_gather(cdfbuf, [above, lane])
                e0 = plsc.load_gather(ebbuf, [below, crow])
                e1 = plsc.load_gather(ebbuf, [above, crow])
                uu = ((2 * j + 1).astype(_f32)) * _f32(1.0 / (2 * J))
                denom = c1 - c0
                denom = jnp.where(denom < 1e-5, 1.0, denom)
                t = jnp.clip((uu - c0) / denom, 0.0, 1.0)
                binsv = e0 + t * (e1 - e0)
                eucl = NEAR + binsv * (FAR - NEAR)
                binsbuf[j, pl.ds(col0, 16)] = binsv
                euclbuf[j, pl.ds(col0, 16)] = eucl
                return run

            mbuf[J, :] = zi
            return 0
        lax.fori_loop(0, G, group_body, 0)

        pltpu.sync_copy(euclbuf.at[pl.ds(0, NO)], bs_hbm.at[:, pl.ds(base, C)])
        pltpu.sync_copy(euclbuf.at[pl.ds(1, NO)], be_hbm.at[:, pl.ds(base, C)])
        pltpu.sync_copy(binsbuf.at[pl.ds(0, NO)], ss_hbm.at[:, pl.ds(base, C)])
        pltpu.sync_copy(binsbuf.at[pl.ds(1, NO)], se_hbm.at[:, pl.ds(base, C)])
        return 0
    lax.fori_loop(0, CHUNKS, chunk_body, 0)


_sampler = functools.partial(
    pl.kernel,
    mesh=_mesh,
    compiler_params=pltpu.CompilerParams(
        needs_layout_passes=False, use_tc_tiling_on_sc=False),
    out_type=[jax.ShapeDtypeStruct((NO, B), _f32)] * 4,
    scratch_types=[
        pltpu.VMEM((C * N,), _f32),
        pltpu.VMEM((NB, C), _f32),
        pltpu.VMEM((NB, 16), _f32),
        pltpu.VMEM((J + 1, 16), _i32),
        pltpu.VMEM((J, C), _f32),
        pltpu.VMEM((J, C), _f32),
    ],
)(_body)


def kernel(weights, existing_bins):
    wf = weights.reshape(B * N)
    ebt = jnp.transpose(existing_bins)
    bs, be, ss, se = _sampler(wf, ebt)
    def _t(x):
        return jnp.transpose(x)[:, :, None]
    return (_t(bs), _t(be), _t(ss), _t(se))


# speedup vs baseline: 15.5137x; 1.1037x over previous
import functools

import jax
import jax.numpy as jnp
from jax import lax
from jax.experimental import pallas as pl
from jax.experimental.pallas import tpu as pltpu
from jax.experimental.pallas import tpu_sc as plsc

B = 16384
N = 128
NB = N + 1
J = 65
NO = J - 1
EPS = 1e-5
NEAR = 0.05
FAR = 6.0

NUM_CORES = 2
NUM_SUBCORES = 16
NW = NUM_CORES * NUM_SUBCORES
RAYS_PER_W = B // NW
C = 128
G = C // 16
CHUNKS = RAYS_PER_W // C

_mesh = plsc.VectorSubcoreMesh(core_axis_name="c", subcore_axis_name="s")

_f32 = jnp.float32
_i32 = jnp.int32


def _body(w_hbm, ebt_hbm,
          bs_hbm, be_hbm, ss_hbm, se_hbm,
          wbuf0, wbuf1, ebbuf0, ebbuf1, cdfbuf, mbuf,
          binsbuf0, binsbuf1, euclbuf0, euclbuf1,
          sem_w0, sem_w1, sem_e0, sem_e1, sem_o0, sem_o1):
    wbufs = [wbuf0, wbuf1]
    ebbufs = [ebbuf0, ebbuf1]
    binsbufs = [binsbuf0, binsbuf1]
    euclbufs = [euclbuf0, euclbuf1]
    sem_w = [sem_w0, sem_w1]
    sem_e = [sem_e0, sem_e1]
    sem_o = [sem_o0, sem_o1]

    wid = lax.axis_index("s") * NUM_CORES + lax.axis_index("c")
    lane = lax.broadcasted_iota(_i32, (16,), 0)
    zf = jnp.zeros((16,), _f32)
    zi = jnp.zeros((16,), _i32)
    ones_i = jnp.ones((16,), _i32)

    @plsc.parallel_loop(0, J + 1, unroll=6)
    def _minit(j):
        mbuf[j, :] = zi

    def start_in(ci):
        slot = ci % 2
        base = wid * RAYS_PER_W + ci * C
        hw = pltpu.async_copy(w_hbm.at[pl.ds(base * N, C * N)],
                              wbufs[slot], sem_w[slot])
        he = pltpu.async_copy(ebt_hbm.at[:, pl.ds(base, C)],
                              ebbufs[slot], sem_e[slot])
        return hw, he

    hin = {0: start_in(0), 1: start_in(1)}
    hout = {}
    for ci in range(CHUNKS):
        slot = ci % 2
        hw, he = hin[ci]
        hw.wait()
        he.wait()
        if ci >= 2:
            for h in hout[ci - 2]:
                h.wait()
        wbuf = wbufs[slot]
        ebbuf = ebbufs[slot]
        binsbuf = binsbufs[slot]
        euclbuf = euclbufs[slot]

        def group_body(g, _, wbuf=wbuf, ebbuf=ebbuf,
                       binsbuf=binsbuf, euclbuf=euclbuf):
            crow = g * 16 + lane
            wbase = crow * N

            cdfbuf[0, :] = zf

            @plsc.parallel_loop(0, N, step=8, unroll=2, carry=zf)
            def total(i, acc):
                ib = wbase + i
                w = [plsc.load_gather(wbuf, [ib + k]) for k in range(8)]
                s01 = w[0] + w[1]
                s23 = w[2] + w[3]
                s45 = w[4] + w[5]
                s67 = w[6] + w[7]
                s03 = s01 + s23
                s47 = s45 + s67
                p = [w[0], s01, s01 + w[2], s03, s03 + w[4], s03 + s45,
                     s03 + s45 + w[6], s03 + s47]
                for k in range(8):
                    cdfbuf[i + 1 + k, :] = acc + p[k]
                return acc + p[7]

            pad = jnp.maximum(EPS - total, 0.0)
            inv = 1.0 / (total + pad)
            padper = pad * (1.0 / N)

            @plsc.parallel_loop(0, N, unroll=8)
            def _pb(i):
                raw = cdfbuf[i + 1, :]
                fi = (i + 1).astype(_f32)
                c = jnp.minimum((raw + padper * fi) * inv, 1.0)
                cdfbuf[i + 1, :] = c
                k = (c * float(J) + 0.5).astype(_i32)
                plsc.addupdate_scatter(mbuf, [k, lane], ones_i)

            col0 = g * 16

            @plsc.parallel_loop(0, J, unroll=5, carry=zi)
            def _run(j, run):
                run = run + mbuf[j, :]
                mbuf[j, :] = zi
                below = run
                above = below + 1
                c0 = plsc.load_gather(cdfbuf, [below, lane])
                c1 = plsc.load_gather(cdfbuf, [above, lane])
                e0 = plsc.load_gather(ebbuf, [below, crow])
                e1 = plsc.load_gather(ebbuf, [above, crow])
                uu = ((2 * j + 1).astype(_f32)) * _f32(1.0 / (2 * J))
                denom = c1 - c0
                denom = jnp.where(denom < 1e-5, 1.0, denom)
                t = jnp.clip((uu - c0) / denom, 0.0, 1.0)
                binsv = e0 + t * (e1 - e0)
                eucl = NEAR + binsv * (FAR - NEAR)
                binsbuf[j, pl.ds(col0, 16)] = binsv
                euclbuf[j, pl.ds(col0, 16)] = eucl
                return run

            mbuf[J, :] = zi
            return 0
        lax.fori_loop(0, G, group_body, 0)

        base = wid * RAYS_PER_W + ci * C
        s = sem_o[slot]
        hout[ci] = [
            pltpu.async_copy(euclbuf.at[pl.ds(0, NO)],
                             bs_hbm.at[:, pl.ds(base, C)], s),
            pltpu.async_copy(euclbuf.at[pl.ds(1, NO)],
                             be_hbm.at[:, pl.ds(base, C)], s),
            pltpu.async_copy(binsbuf.at[pl.ds(0, NO)],
                             ss_hbm.at[:, pl.ds(base, C)], s),
            pltpu.async_copy(binsbuf.at[pl.ds(1, NO)],
                             se_hbm.at[:, pl.ds(base, C)], s),
        ]
        if ci + 2 < CHUNKS:
            hin[ci + 2] = start_in(ci + 2)
    for ci in (CHUNKS - 2, CHUNKS - 1):
        for h in hout[ci]:
            h.wait()


_sampler = functools.partial(
    pl.kernel,
    mesh=_mesh,
    compiler_params=pltpu.CompilerParams(
        needs_layout_passes=False, use_tc_tiling_on_sc=False),
    out_type=[jax.ShapeDtypeStruct((NO, B), _f32)] * 4,
    scratch_types=[
        pltpu.VMEM((C * N,), _f32),
        pltpu.VMEM((C * N,), _f32),
        pltpu.VMEM((NB, C), _f32),
        pltpu.VMEM((NB, C), _f32),
        pltpu.VMEM((NB, 16), _f32),
        pltpu.VMEM((J + 1, 16), _i32),
        pltpu.VMEM((J, C), _f32),
        pltpu.VMEM((J, C), _f32),
        pltpu.VMEM((J, C), _f32),
        pltpu.VMEM((J, C), _f32),
        pltpu.SemaphoreType.DMA,
        pltpu.SemaphoreType.DMA,
        pltpu.SemaphoreType.DMA,
        pltpu.SemaphoreType.DMA,
        pltpu.SemaphoreType.DMA,
        pltpu.SemaphoreType.DMA,
    ],
)(_body)


def kernel(weights, existing_bins):
    wf = weights.reshape(B * N)
    ebt = jnp.transpose(existing_bins)
    bs, be, ss, se = _sampler(wf, ebt)
    def _t(x):
        return jnp.transpose(x)[:, :, None]
    return (_t(bs), _t(be), _t(ss), _t(se))
